# baseline (speedup 1.0000x reference)
import functools

import jax
import jax.numpy as jnp
from jax.experimental import pallas as pl
from jax.experimental.pallas import tpu as pltpu


_M = 4
_W = 128


def _drq_kernel(scale_ref, x_ref, cb_ref, codes_ref, loss_ref,
                tab_ref, cbm_ref, cn_ref):
    i = pl.program_id(0)
    k, d = cb_ref.shape

    @pl.when(i == 0)
    def _init():
        cb = cb_ref[...]
        pad = jnp.zeros((k, _W - d - 1), jnp.float32)
        ones = jnp.ones((k, 1), jnp.float32)
        for m in range(_M):
            cbm = cb * scale_ref[m]
            cbm_ref[m] = cbm
            cn = jnp.sum(cbm * cbm, axis=1)
            cn_ref[m] = jnp.broadcast_to(cn[None, :], (8, k))
            tab_ref[m] = jnp.concatenate([cbm, ones, pad], axis=1)
        loss_ref[0] = 0.0

    x = x_ref[...]
    bn = x.shape[0]
    n_total = pl.num_programs(0) * bn
    inv_nd = 1.0 / (n_total * d)

    dot = functools.partial(
        jax.lax.dot_general, preferred_element_type=jnp.float32)

    residual = x
    qsoft = jnp.zeros_like(x)
    qhard = jnp.zeros_like(x)
    part = jnp.float32(0.0)
    for m in range(_M):
        tab = tab_ref[m]
        cbm = cbm_ref[m]
        cnl = cn_ref[m, 0:1, :]
        rn = jnp.sum(residual * residual, axis=1, keepdims=True)
        g2 = dot(residual + residual, cbm,
                 (((1,), (1,)), ((), ())))
        nd = (rn - g2) + cnl
        mn = jnp.min(nd, axis=1, keepdims=True)
        e = jnp.exp(mn - nd)
        se = dot(e, tab, (((1,), (0,)), ((), ())))
        soft = se[:, :d] / se[:, d:d + 1]
        idxs = jax.lax.broadcasted_iota(jnp.int32, nd.shape, 1)
        masked = jnp.where(nd <= mn, idxs, jnp.int32(k))
        code = jnp.min(masked, axis=1)
        codes_ref[:, m] = code
        oh = (idxs == code[:, None]).astype(jnp.float32)
        hard = dot(oh, tab, (((1,), (0,)), ((), ())))[:, :d]
        residual = residual - hard
        qsoft = qsoft + soft
        qhard = qhard + hard
        part += 0.1 * jnp.sum((x - qsoft) ** 2) + jnp.sum((x - qhard) ** 2)
    part += 0.1 * jnp.sum((qsoft - qhard) ** 2)

    loss_ref[0] += part * inv_nd


def kernel(x, codebook, scale):
    n, d = x.shape
    k = codebook.shape[0]
    bn = 512
    grid = (n // bn,)
    codes, loss = pl.pallas_call(
        _drq_kernel,
        grid=grid,
        in_specs=[
            pl.BlockSpec(memory_space=pltpu.SMEM),
            pl.BlockSpec((bn, d), lambda i: (i, 0)),
            pl.BlockSpec((k, d), lambda i: (0, 0)),
        ],
        out_specs=[
            pl.BlockSpec((bn, _M), lambda i: (i, 0)),
            pl.BlockSpec(memory_space=pltpu.SMEM),
        ],
        out_shape=[
            jax.ShapeDtypeStruct((n, _M), jnp.int32),
            jax.ShapeDtypeStruct((1,), jnp.float32),
        ],
        scratch_shapes=[
            pltpu.VMEM((_M, k, _W), jnp.float32),
            pltpu.VMEM((_M, k, d), jnp.float32),
            pltpu.VMEM((_M, 8, k), jnp.float32),
        ],
        compiler_params=pltpu.CompilerParams(
            dimension_semantics=("arbitrary",)),
    )(scale, x, codebook)
    return codes, loss[0]


# speedup vs baseline: 1.1674x; 1.1674x over previous
import functools

import jax
import jax.numpy as jnp
from jax.experimental import pallas as pl
from jax.experimental.pallas import tpu as pltpu


_M = 4
_W = 128


def _bf16_exact_split(v):
    b0 = v.astype(jnp.bfloat16).astype(jnp.float32)
    r0 = v - b0
    b1 = r0.astype(jnp.bfloat16).astype(jnp.float32)
    b2 = (r0 - b1).astype(jnp.bfloat16).astype(jnp.float32)
    return b0, b1, b2


def _drq_kernel(scale_ref, x_ref, cb_ref, codes_ref, loss_ref, tab_ref):
    i = pl.program_id(0)
    k, d = cb_ref.shape

    @pl.when(i == 0)
    def _init():
        cb = cb_ref[...]
        pad = jnp.zeros((k, _W - d - 4), jnp.float32)
        ones = jnp.ones((k, 1), jnp.float32)
        for m in range(_M):
            cbm = cb * scale_ref[m]
            cn = jnp.sum(cbm * cbm, axis=1, keepdims=True)
            c0, c1, c2 = _bf16_exact_split(cn)
            tab_ref[m] = jnp.concatenate(
                [cbm, c0, c1, c2, ones, pad], axis=1)
        loss_ref[0] = 0.0

    x = x_ref[...]
    bn = x.shape[0]
    n_total = pl.num_programs(0) * bn
    inv_nd = 1.0 / (n_total * d)

    dot = functools.partial(
        jax.lax.dot_general, preferred_element_type=jnp.float32)

    neg1 = jnp.full((bn, 3), -1.0, jnp.float32)
    rpad = jnp.zeros((bn, _W - d - 3), jnp.float32)
    idxs = jax.lax.broadcasted_iota(jnp.int32, (bn, k), 1)

    residual = x
    qsoft = jnp.zeros_like(x)
    qhard = jnp.zeros_like(x)
    part = jnp.float32(0.0)
    for m in range(_M):
        tab = tab_ref[m]
        r_aug = jnp.concatenate([residual + residual, neg1, rpad], axis=1)
        logits = dot(r_aug, tab, (((1,), (1,)), ((), ())))
        mx = jnp.max(logits, axis=1, keepdims=True)
        e = jnp.exp(logits - mx)
        se = dot(e, tab, (((1,), (0,)), ((), ())))
        soft = se[:, :d] / se[:, d + 3:d + 4]
        mask = logits >= mx
        code = jnp.min(jnp.where(mask, idxs, jnp.int32(k)), axis=1)
        codes_ref[:, m] = code
        hv = dot(mask.astype(jnp.float32), tab,
                 (((1,), (0,)), ((), ())))
        hard = hv[:, :d] / hv[:, d + 3:d + 4]
        residual = residual - hard
        qsoft = qsoft + soft
        qhard = qhard + hard
        part += 0.1 * jnp.sum((x - qsoft) ** 2) + jnp.sum((x - qhard) ** 2)
    part += 0.1 * jnp.sum((qsoft - qhard) ** 2)

    loss_ref[0] += part * inv_nd


def kernel(x, codebook, scale):
    n, d = x.shape
    k = codebook.shape[0]
    bn = 512
    grid = (n // bn,)
    codes, loss = pl.pallas_call(
        _drq_kernel,
        grid=grid,
        in_specs=[
            pl.BlockSpec(memory_space=pltpu.SMEM),
            pl.BlockSpec((bn, d), lambda i: (i, 0)),
            pl.BlockSpec((k, d), lambda i: (0, 0)),
        ],
        out_specs=[
            pl.BlockSpec((bn, _M), lambda i: (i, 0)),
            pl.BlockSpec(memory_space=pltpu.SMEM),
        ],
        out_shape=[
            jax.ShapeDtypeStruct((n, _M), jnp.int32),
            jax.ShapeDtypeStruct((1,), jnp.float32),
        ],
        scratch_shapes=[pltpu.VMEM((_M, k, _W), jnp.float32)],
        compiler_params=pltpu.CompilerParams(
            dimension_semantics=("arbitrary",)),
    )(scale, x, codebook)
    return codes, loss[0]


# speedup vs baseline: 1.2023x; 1.0299x over previous
import functools

import jax
import jax.numpy as jnp
from jax.experimental import pallas as pl
from jax.experimental.pallas import tpu as pltpu


_M = 4
_W = 128


def _bf16_split(v, n):
    out = []
    for _ in range(n):
        b = v.astype(jnp.bfloat16).astype(jnp.float32)
        out.append(b)
        v = v - b
    return out


def _drq_kernel(scale_ref, x_ref, cb_ref, codes_ref, loss_ref, tab_ref):
    i = pl.program_id(0)
    k, d = cb_ref.shape

    @pl.when(i == 0)
    def _init():
        cb = cb_ref[...]
        pad = jnp.zeros((k, _W - d - 6), jnp.float32)
        ones = jnp.ones((k, 1), jnp.float32)
        iota = jax.lax.broadcasted_iota(
            jnp.int32, (k, 1), 0).astype(jnp.float32)
        i0, i1 = _bf16_split(iota, 2)
        for m in range(_M):
            cbm = cb * scale_ref[m]
            cn = jnp.sum(cbm * cbm, axis=1, keepdims=True)
            c0, c1, c2 = _bf16_split(cn, 3)
            tab_ref[m] = jnp.concatenate(
                [cbm, c0, c1, c2, ones, i0, i1, pad],
                axis=1).astype(jnp.bfloat16)
        loss_ref[0] = 0.0

    x = x_ref[...]
    bn = x.shape[0]
    n_total = pl.num_programs(0) * bn
    inv_nd = 1.0 / (n_total * d)

    dot = functools.partial(
        jax.lax.dot_general, preferred_element_type=jnp.float32)

    neg1 = jnp.full((bn, 3), -1.0, jnp.float32)
    rpad = jnp.zeros((bn, _W - d - 3), jnp.float32)

    residual = x
    qsoft = jnp.zeros_like(x)
    qhard = jnp.zeros_like(x)
    part = jnp.float32(0.0)
    for m in range(_M):
        tab = tab_ref[m]
        r_aug = jnp.concatenate(
            [residual + residual, neg1, rpad], axis=1).astype(jnp.bfloat16)
        logits = dot(r_aug, tab, (((1,), (1,)), ((), ())))
        mx = jnp.max(logits, axis=1, keepdims=True)
        e = jnp.exp(logits - mx).astype(jnp.bfloat16)
        se = dot(e, tab, (((1,), (0,)), ((), ())))
        soft = se[:, :d] / se[:, d + 3:d + 4]
        mask = (logits >= mx).astype(jnp.bfloat16)
        hv = dot(mask, tab, (((1,), (0,)), ((), ())))
        cnt = hv[:, d + 3:d + 4]
        code_f = (hv[:, d + 4:d + 5] + hv[:, d + 5:d + 6]) / cnt
        codes_ref[:, m] = code_f[:, 0].astype(jnp.int32)
        hard = hv[:, :d] / cnt
        residual = residual - hard
        qsoft = qsoft + soft
        qhard = qhard + hard
        part += 0.1 * jnp.sum((x - qsoft) ** 2) + jnp.sum((x - qhard) ** 2)
    part += 0.1 * jnp.sum((qsoft - qhard) ** 2)

    loss_ref[0] += part * inv_nd


def kernel(x, codebook, scale):
    n, d = x.shape
    k = codebook.shape[0]
    bn = 512
    grid = (n // bn,)
    codes, loss = pl.pallas_call(
        _drq_kernel,
        grid=grid,
        in_specs=[
            pl.BlockSpec(memory_space=pltpu.SMEM),
            pl.BlockSpec((bn, d), lambda i: (i, 0)),
            pl.BlockSpec((k, d), lambda i: (0, 0)),
        ],
        out_specs=[
            pl.BlockSpec((bn, _M), lambda i: (i, 0)),
            pl.BlockSpec(memory_space=pltpu.SMEM),
        ],
        out_shape=[
            jax.ShapeDtypeStruct((n, _M), jnp.int32),
            jax.ShapeDtypeStruct((1,), jnp.float32),
        ],
        scratch_shapes=[pltpu.VMEM((_M, k, _W), jnp.bfloat16)],
        compiler_params=pltpu.CompilerParams(
            dimension_semantics=("arbitrary",)),
    )(scale, x, codebook)
    return codes, loss[0]
